# baseline (speedup 1.0000x reference)
import jax
import jax.numpy as jnp
from jax.experimental import pallas as pl
from jax.experimental.pallas import tpu as pltpu

B = 1024
IN_DIM = 512
D_MODEL = 1024
D_FF = 4096
N_LAYERS = 3
OUT_DIM = 128

FF_BLK = 1024
NFF = D_FF // FF_BLK
D_BLK = 256
ND = D_MODEL // D_BLK
NFB = NFF + ND
NSTAGE = N_LAYERS + 1


def _dot(a, b, dims):
    return jax.lax.dot_general(a, b, (dims, ((), ())),
                               preferred_element_type=jnp.float32)


def _gelu_exact(x):
    return 0.5 * x * (1.0 + jax.lax.erf(x * 0.7071067811865476))


def _ln(a, g, b, eps=1e-5):
    m = jnp.mean(a, axis=-1, keepdims=True)
    msq = jnp.mean(a * a, axis=-1, keepdims=True)
    r = jax.lax.rsqrt(jnp.maximum(msq - m * m, 0.0) + eps)
    scale = r * g
    return a * scale + (b - m * scale)


def _fold_body(Wv_ref, Wo_ref, bv_ref, bo_ref, WvoT_ref, bvo_ref):
    WvoT_ref[...] = _dot(Wv_ref[...], Wo_ref[...], ((0,), (1,)))
    bvo_ref[...] = _dot(bv_ref[...], Wo_ref[...], ((1,), (1,))) + bo_ref[...]


def _main_body(x_ref, Win_ref, bin_ref, WvoT_ref, bvo_ref,
               W1_ref, b1_ref, W2_ref, b2_ref,
               g1_ref, be1_ref, g2_ref, be2_ref, gf_ref, bf_ref,
               Wh1_ref, bh1_ref, Wh2_ref, bh2_ref,
               out_ref, h_s, h1_s, gc_s):
    l = pl.program_id(0)
    c = pl.program_id(1)

    @pl.when((l == 0) & (c == 0))
    def _():
        h_s[...] = _dot(x_ref[...], Win_ref[...], ((1,), (1,))) + bin_ref[...]

    @pl.when((l < N_LAYERS) & (c == 0))
    def _():
        h = h_s[...]
        a = h + _dot(h, WvoT_ref[...], ((1,), (0,))) + bvo_ref[...]
        h1_s[...] = _ln(a, g1_ref[0], be1_ref[0])

    @pl.when((l < N_LAYERS) & (c < NFF))
    def _():
        gc = _gelu_exact(_dot(h1_s[...], W1_ref[0], ((1,), (1,))) + b1_ref[0])
        gc_s[:, pl.ds(c * FF_BLK, FF_BLK)] = gc.astype(jnp.bfloat16)

    @pl.when((l < N_LAYERS) & (c >= NFF))
    def _():
        d = c - NFF
        part = _dot(gc_s[...], W2_ref[0].astype(jnp.bfloat16), ((1,), (1,)))
        sl = pl.ds(d * D_BLK, D_BLK)
        h_s[:, sl] = h1_s[:, sl] + part + b2_ref[0, :, sl]

    @pl.when((l < N_LAYERS) & (c == NFB - 1))
    def _():
        h2 = _ln(h_s[...], g2_ref[0], be2_ref[0])
        h_s[...] = h2

        @pl.when(l == N_LAYERS - 1)
        def _():
            h_s[...] = _ln(h2, gf_ref[...], bf_ref[...])

    @pl.when((l == N_LAYERS) & (c < NFF))
    def _():
        gc = _gelu_exact(_dot(h_s[...], Wh1_ref[0], ((1,), (1,))) + bh1_ref[0])
        gc_s[:, pl.ds(c * FF_BLK, FF_BLK)] = gc.astype(jnp.bfloat16)

    @pl.when((l == N_LAYERS) & (c == NFF))
    def _():
        out_ref[...] = (_dot(gc_s[...], Wh2_ref[...].astype(jnp.bfloat16),
                             ((1,), (1,)))
                        + bh2_ref[...])


def kernel(x, W_in, b_in, Wq, bq, Wk, bk, Wv, bv, Wo, bo,
           W1, b1, W2, b2, g1, be1, g2, be2, gf, bf,
           Wout1, bout1, Wout2, bout2):
    f32 = jnp.float32
    bv2 = bv.reshape(1, D_MODEL)
    bo2 = bo.reshape(1, D_MODEL)
    bin2 = b_in.reshape(1, D_MODEL)
    b1r = b1.reshape(N_LAYERS, 1, D_FF)
    b2r = b2.reshape(N_LAYERS, 1, D_MODEL)
    g1r = g1.reshape(N_LAYERS, 1, D_MODEL)
    be1r = be1.reshape(N_LAYERS, 1, D_MODEL)
    g2r = g2.reshape(N_LAYERS, 1, D_MODEL)
    be2r = be2.reshape(N_LAYERS, 1, D_MODEL)
    gf2 = gf.reshape(1, D_MODEL)
    bf2 = bf.reshape(1, D_MODEL)
    Wh1 = Wout1.reshape(1, D_FF, D_MODEL)
    bh1 = bout1.reshape(1, 1, D_FF)
    bh2 = bout2.reshape(1, OUT_DIM)

    WvoT, bvo = pl.pallas_call(
        _fold_body,
        out_shape=[jax.ShapeDtypeStruct((D_MODEL, D_MODEL), f32),
                   jax.ShapeDtypeStruct((1, D_MODEL), f32)],
    )(Wv, Wo, bv2, bo2)

    lidx = lambda l: jnp.minimum(l, N_LAYERS - 1)

    out = pl.pallas_call(
        _main_body,
        grid=(NSTAGE, NFB),
        in_specs=[
            pl.BlockSpec((B, IN_DIM), lambda l, c: (0, 0)),
            pl.BlockSpec((D_MODEL, IN_DIM), lambda l, c: (0, 0)),
            pl.BlockSpec((1, D_MODEL), lambda l, c: (0, 0)),
            pl.BlockSpec((D_MODEL, D_MODEL), lambda l, c: (0, 0)),
            pl.BlockSpec((1, D_MODEL), lambda l, c: (0, 0)),
            pl.BlockSpec((1, FF_BLK, D_MODEL),
                         lambda l, c: (lidx(l), jnp.minimum(c, NFF - 1), 0)),
            pl.BlockSpec((1, 1, FF_BLK),
                         lambda l, c: (lidx(l), 0, jnp.minimum(c, NFF - 1))),
            pl.BlockSpec((1, D_BLK, D_FF),
                         lambda l, c: (lidx(l), jnp.maximum(c - NFF, 0), 0)),
            pl.BlockSpec((1, 1, D_MODEL), lambda l, c: (lidx(l), 0, 0)),
            pl.BlockSpec((1, 1, D_MODEL), lambda l, c: (lidx(l), 0, 0)),
            pl.BlockSpec((1, 1, D_MODEL), lambda l, c: (lidx(l), 0, 0)),
            pl.BlockSpec((1, 1, D_MODEL), lambda l, c: (lidx(l), 0, 0)),
            pl.BlockSpec((1, 1, D_MODEL), lambda l, c: (lidx(l), 0, 0)),
            pl.BlockSpec((1, D_MODEL), lambda l, c: (0, 0)),
            pl.BlockSpec((1, D_MODEL), lambda l, c: (0, 0)),
            pl.BlockSpec((1, FF_BLK, D_MODEL),
                         lambda l, c: (0,
                                       jnp.where(l == N_LAYERS,
                                                 jnp.minimum(c, NFF - 1), 0),
                                       0)),
            pl.BlockSpec((1, 1, FF_BLK),
                         lambda l, c: (0, 0,
                                       jnp.where(l == N_LAYERS,
                                                 jnp.minimum(c, NFF - 1), 0))),
            pl.BlockSpec((OUT_DIM, D_FF), lambda l, c: (0, 0)),
            pl.BlockSpec((1, OUT_DIM), lambda l, c: (0, 0)),
        ],
        out_specs=pl.BlockSpec((B, OUT_DIM), lambda l, c: (0, 0)),
        out_shape=jax.ShapeDtypeStruct((B, OUT_DIM), f32),
        scratch_shapes=[pltpu.VMEM((B, D_MODEL), f32),
                        pltpu.VMEM((B, D_MODEL), f32),
                        pltpu.VMEM((B, D_FF), jnp.bfloat16)],
        compiler_params=pltpu.CompilerParams(
            dimension_semantics=("arbitrary", "arbitrary")),
    )(x, W_in, bin2, WvoT, bvo, W1, b1r, W2, b2r,
      g1r, be1r, g2r, be2r, gf2, bf2, Wh1, bh1, Wout2, bh2)

    return out


# speedup vs baseline: 1.0435x; 1.0435x over previous
import jax
import jax.numpy as jnp
from jax.experimental import pallas as pl
from jax.experimental.pallas import tpu as pltpu

B = 1024
IN_DIM = 512
D_MODEL = 1024
D_FF = 4096
N_LAYERS = 3
OUT_DIM = 128

FF_BLK = 1024
NFF = D_FF // FF_BLK
D_BLK = 256
ND = D_MODEL // D_BLK
NFB = NFF + ND

BMH = 512
FFH = 2048
NBH = B // BMH
NFH = D_FF // FFH


def _dot(a, b, dims):
    return jax.lax.dot_general(a, b, (dims, ((), ())),
                               preferred_element_type=jnp.float32)


def _gelu_exact(x):
    return 0.5 * x * (1.0 + jax.lax.erf(x * 0.7071067811865476))


def _ln(a, g, b, eps=1e-5):
    m = jnp.mean(a, axis=-1, keepdims=True)
    msq = jnp.mean(a * a, axis=-1, keepdims=True)
    r = jax.lax.rsqrt(jnp.maximum(msq - m * m, 0.0) + eps)
    scale = r * g
    return a * scale + (b - m * scale)


def _main_body(x_ref, Win_ref, bin_ref, Wv_ref, Wo_ref, bv_ref, bo_ref,
               W1_ref, b1_ref, W2_ref, b2_ref,
               g1_ref, be1_ref, g2_ref, be2_ref, gf_ref, bf_ref,
               out_ref, h_s, gc_s, WvoT_s, bvo_s):
    l = pl.program_id(0)
    c = pl.program_id(1)

    @pl.when((l == 0) & (c == 0))
    def _():
        WvoT_s[...] = _dot(Wv_ref[...], Wo_ref[...], ((0,), (1,)))
        bvo_s[...] = _dot(bv_ref[...], Wo_ref[...], ((1,), (1,))) + bo_ref[...]
        h_s[...] = _dot(x_ref[...], Win_ref[...], ((1,), (1,))) + bin_ref[...]

    @pl.when(c == 0)
    def _():
        h = h_s[...]
        a = h + _dot(h, WvoT_s[...], ((1,), (0,))) + bvo_s[...]
        h_s[...] = _ln(a, g1_ref[0], be1_ref[0])

    @pl.when(c < NFF)
    def _():
        gc = _gelu_exact(_dot(h_s[...], W1_ref[0], ((1,), (1,))) + b1_ref[0])
        gc_s[:, pl.ds(c * FF_BLK, FF_BLK)] = gc.astype(jnp.bfloat16)

    @pl.when(c >= NFF)
    def _():
        d = c - NFF
        part = _dot(gc_s[...], W2_ref[0].astype(jnp.bfloat16), ((1,), (1,)))
        sl = pl.ds(d * D_BLK, D_BLK)
        h_s[:, sl] = h_s[:, sl] + part + b2_ref[0, :, sl]

    @pl.when(c == NFB - 1)
    def _():
        h_s[...] = _ln(h_s[...], g2_ref[0], be2_ref[0])

    @pl.when((c == NFB - 1) & (l == N_LAYERS - 1))
    def _():
        out_ref[...] = _ln(h_s[...], gf_ref[...], bf_ref[...])


def _head_body(hf_ref, W1_ref, b1_ref, W2_ref, b2_ref, out_ref):
    c = pl.program_id(1)
    g = _gelu_exact(_dot(hf_ref[...], W1_ref[...], ((1,), (1,))) + b1_ref[...])
    part = _dot(g, W2_ref[...], ((1,), (1,)))

    @pl.when(c == 0)
    def _():
        out_ref[...] = part + b2_ref[...]

    @pl.when(c > 0)
    def _():
        out_ref[...] += part


def kernel(x, W_in, b_in, Wq, bq, Wk, bk, Wv, bv, Wo, bo,
           W1, b1, W2, b2, g1, be1, g2, be2, gf, bf,
           Wout1, bout1, Wout2, bout2):
    f32 = jnp.float32
    bv2 = bv.reshape(1, D_MODEL)
    bo2 = bo.reshape(1, D_MODEL)
    bin2 = b_in.reshape(1, D_MODEL)
    b1r = b1.reshape(N_LAYERS, 1, D_FF)
    b2r = b2.reshape(N_LAYERS, 1, D_MODEL)
    g1r = g1.reshape(N_LAYERS, 1, D_MODEL)
    be1r = be1.reshape(N_LAYERS, 1, D_MODEL)
    g2r = g2.reshape(N_LAYERS, 1, D_MODEL)
    be2r = be2.reshape(N_LAYERS, 1, D_MODEL)
    gf2 = gf.reshape(1, D_MODEL)
    bf2 = bf.reshape(1, D_MODEL)
    bout1r = bout1.reshape(1, D_FF)
    bout2r = bout2.reshape(1, OUT_DIM)

    hf = pl.pallas_call(
        _main_body,
        grid=(N_LAYERS, NFB),
        in_specs=[
            pl.BlockSpec((B, IN_DIM), lambda l, c: (0, 0)),
            pl.BlockSpec((D_MODEL, IN_DIM), lambda l, c: (0, 0)),
            pl.BlockSpec((1, D_MODEL), lambda l, c: (0, 0)),
            pl.BlockSpec((D_MODEL, D_MODEL), lambda l, c: (0, 0)),
            pl.BlockSpec((D_MODEL, D_MODEL), lambda l, c: (0, 0)),
            pl.BlockSpec((1, D_MODEL), lambda l, c: (0, 0)),
            pl.BlockSpec((1, D_MODEL), lambda l, c: (0, 0)),
            pl.BlockSpec((1, FF_BLK, D_MODEL),
                         lambda l, c: (l, jnp.minimum(c, NFF - 1), 0)),
            pl.BlockSpec((1, 1, FF_BLK),
                         lambda l, c: (l, 0, jnp.minimum(c, NFF - 1))),
            pl.BlockSpec((1, D_BLK, D_FF),
                         lambda l, c: (l, jnp.maximum(c - NFF, 0), 0)),
            pl.BlockSpec((1, 1, D_MODEL), lambda l, c: (l, 0, 0)),
            pl.BlockSpec((1, 1, D_MODEL), lambda l, c: (l, 0, 0)),
            pl.BlockSpec((1, 1, D_MODEL), lambda l, c: (l, 0, 0)),
            pl.BlockSpec((1, 1, D_MODEL), lambda l, c: (l, 0, 0)),
            pl.BlockSpec((1, 1, D_MODEL), lambda l, c: (l, 0, 0)),
            pl.BlockSpec((1, D_MODEL), lambda l, c: (0, 0)),
            pl.BlockSpec((1, D_MODEL), lambda l, c: (0, 0)),
        ],
        out_specs=pl.BlockSpec((B, D_MODEL), lambda l, c: (0, 0)),
        out_shape=jax.ShapeDtypeStruct((B, D_MODEL), f32),
        scratch_shapes=[pltpu.VMEM((B, D_MODEL), f32),
                        pltpu.VMEM((B, D_FF), jnp.bfloat16),
                        pltpu.VMEM((D_MODEL, D_MODEL), f32),
                        pltpu.VMEM((1, D_MODEL), f32)],
        compiler_params=pltpu.CompilerParams(
            dimension_semantics=("arbitrary", "arbitrary")),
    )(x, W_in, bin2, Wv, Wo, bv2, bo2, W1, b1r, W2, b2r,
      g1r, be1r, g2r, be2r, gf2, bf2)

    out = pl.pallas_call(
        _head_body,
        grid=(NBH, NFH),
        in_specs=[
            pl.BlockSpec((BMH, D_MODEL), lambda b, c: (b, 0)),
            pl.BlockSpec((FFH, D_MODEL), lambda b, c: (c, 0)),
            pl.BlockSpec((1, FFH), lambda b, c: (0, c)),
            pl.BlockSpec((OUT_DIM, FFH), lambda b, c: (0, c)),
            pl.BlockSpec((1, OUT_DIM), lambda b, c: (0, 0)),
        ],
        out_specs=pl.BlockSpec((BMH, OUT_DIM), lambda b, c: (b, 0)),
        out_shape=jax.ShapeDtypeStruct((B, OUT_DIM), f32),
        compiler_params=pltpu.CompilerParams(
            dimension_semantics=("parallel", "arbitrary")),
    )(hf, Wout1, bout1r, Wout2, bout2r)

    return out
